# baseline (speedup 1.0000x reference)
import functools

import jax
import jax.numpy as jnp
from jax import lax
from jax.experimental import pallas as pl
from jax.experimental.pallas import tpu as pltpu
from jax.experimental.pallas import tpu_sc as plsc

VOCAB = 1000
DPAD = 1024
N_ROWS = 1024 * 200
NC, NS = 2, 16
NW = NC * NS
ROWS_PER_W = N_ROWS // NW
B, T = 1024, 200
CHUNK = 40
N_CHUNKS = ROWS_PER_W // CHUNK


TAIL = VOCAB - 896
TAIL_OFFS = (0, 16, 32, 48, 64, 80, TAIL - 16)


def _gather_body(table_hbm, idx_hbm, out_hbm, idx_v, rows_v, tail_v, sem):
    wid = lax.axis_index("s") * NC + lax.axis_index("c")
    base = wid * ROWS_PER_W

    def step(g, carry):
        off = base + g * CHUNK
        b = off // T
        t0 = off % T
        pltpu.sync_copy(idx_hbm.at[pl.ds(off, CHUNK)], idx_v)
        pltpu.async_copy(table_hbm.at[idx_v], rows_v, sem).wait()

        def repack(r, c2):
            for c in TAIL_OFFS:
                tail_v[r, pl.ds(c, 16)] = rows_v[r, pl.ds(896 + c, 16)]
            return c2

        lax.fori_loop(0, CHUNK, repack, 0)
        pltpu.sync_copy(
            rows_v.at[:, pl.ds(0, 896)],
            out_hbm.at[b, pl.ds(t0, CHUNK), pl.ds(0, 896)],
        )
        pltpu.sync_copy(
            tail_v, out_hbm.at[b, pl.ds(t0, CHUNK), pl.ds(896, TAIL)]
        )
        return carry

    lax.fori_loop(0, N_CHUNKS, step, 0)


@jax.jit
def kernel(idx, table):
    mesh = plsc.VectorSubcoreMesh(
        core_axis_name="c", subcore_axis_name="s", num_cores=NC, num_subcores=NS
    )
    k = functools.partial(
        pl.kernel,
        out_type=jax.ShapeDtypeStruct((B, T, VOCAB), jnp.float32),
        mesh=mesh,
        scratch_types=[
            pltpu.VMEM((CHUNK,), jnp.int32),
            pltpu.VMEM((CHUNK, DPAD), jnp.float32),
            pltpu.VMEM((CHUNK, TAIL), jnp.float32),
            pltpu.SemaphoreType.DMA,
        ],
    )(_gather_body)
    table_pad = jnp.pad(table, ((0, 0), (0, DPAD - VOCAB)))
    return k(table_pad, idx.reshape(N_ROWS).astype(jnp.int32))


# speedup vs baseline: 1.2098x; 1.2098x over previous
import functools

import jax
import jax.numpy as jnp
from jax import lax
from jax.experimental import pallas as pl
from jax.experimental.pallas import tpu as pltpu
from jax.experimental.pallas import tpu_sc as plsc

VOCAB = 1000
DPAD = 1024
B, T = 1024, 200
N_ROWS = B * T
NC, NS = 2, 16
NW = NC * NS
ROWS_PER_W = N_ROWS // NW
CHUNK = 40
N_CHUNKS = ROWS_PER_W // CHUNK
MAIN = 896
TAIL = VOCAB - MAIN
TAIL_OFFS = (0, 16, 32, 48, 64, 80, TAIL - 16)


def _gather_body(table_hbm, idx_hbm, out_hbm, idx_v, rows0, rows1, tail0,
                 tail1, gsem0, gsem1, om0, om1, ot0, ot1):
    rows = (rows0, rows1)
    tails = (tail0, tail1)
    gsems = (gsem0, gsem1)
    osems = ((om0, ot0), (om1, ot1))

    wid = lax.axis_index("s") * NC + lax.axis_index("c")
    base = wid * ROWS_PER_W
    pltpu.sync_copy(idx_hbm.at[pl.ds(base, ROWS_PER_W)], idx_v)

    def issue_gather(g, p):
        pltpu.async_copy(
            table_hbm.at[idx_v.at[pl.ds(g * CHUNK, CHUNK)]], rows[p], gsems[p]
        )

    def out_slices(g):
        off = base + g * CHUNK
        b = off // T
        t0 = off % T
        return (
            out_hbm.at[b, pl.ds(t0, CHUNK), pl.ds(0, MAIN)],
            out_hbm.at[b, pl.ds(t0, CHUNK), pl.ds(MAIN, TAIL)],
        )

    def issue_out(g, p):
        main_dst, tail_dst = out_slices(g)
        pltpu.async_copy(rows[p].at[:, pl.ds(0, MAIN)], main_dst, osems[p][0])
        pltpu.async_copy(tails[p], tail_dst, osems[p][1])

    def wait_out(g, p):
        main_dst, tail_dst = out_slices(g)
        pltpu.make_async_copy(
            rows[p].at[:, pl.ds(0, MAIN)], main_dst, osems[p][0]
        ).wait()
        pltpu.make_async_copy(tails[p], tail_dst, osems[p][1]).wait()

    def wait_gather(g, p):
        pltpu.make_async_copy(
            table_hbm.at[idx_v.at[pl.ds(g * CHUNK, CHUNK)]], rows[p], gsems[p]
        ).wait()

    def repack(p):
        rv, tv = rows[p], tails[p]

        def body(r, c2):
            for c in TAIL_OFFS:
                tv[r, pl.ds(c, 16)] = rv[r, pl.ds(MAIN + c, 16)]
            return c2

        lax.fori_loop(0, CHUNK, body, 0)

    def half(g, p):
        @pl.when(g >= 1)
        def _():
            wait_out(g - 1, 1 - p)

        @pl.when(g + 1 < N_CHUNKS)
        def _():
            issue_gather(g + 1, 1 - p)

        wait_gather(g, p)
        repack(p)
        issue_out(g, p)

    issue_gather(0, 0)

    def step(i, carry):
        half(2 * i, 0)
        half(2 * i + 1, 1)
        return carry

    lax.fori_loop(0, N_CHUNKS // 2, step, 0)
    wait_out(N_CHUNKS - 1, 1)


@jax.jit
def kernel(idx, table):
    mesh = plsc.VectorSubcoreMesh(
        core_axis_name="c", subcore_axis_name="s", num_cores=NC, num_subcores=NS
    )
    k = functools.partial(
        pl.kernel,
        out_type=jax.ShapeDtypeStruct((B, T, VOCAB), jnp.float32),
        mesh=mesh,
        scratch_types=[
            pltpu.VMEM((ROWS_PER_W,), jnp.int32),
            pltpu.VMEM((CHUNK, DPAD), jnp.float32),
            pltpu.VMEM((CHUNK, DPAD), jnp.float32),
            pltpu.VMEM((CHUNK, TAIL), jnp.float32),
            pltpu.VMEM((CHUNK, TAIL), jnp.float32),
            pltpu.SemaphoreType.DMA,
            pltpu.SemaphoreType.DMA,
            pltpu.SemaphoreType.DMA,
            pltpu.SemaphoreType.DMA,
            pltpu.SemaphoreType.DMA,
            pltpu.SemaphoreType.DMA,
        ],
    )(_gather_body)
    table_pad = jnp.pad(table, ((0, 0), (0, DPAD - VOCAB)))
    return k(table_pad, idx.reshape(N_ROWS).astype(jnp.int32))
